# baseline (speedup 1.0000x reference)
import functools

import jax
import jax.numpy as jnp
from jax import lax
from jax.experimental import pallas as pl
from jax.experimental.pallas import tpu as pltpu
from jax.experimental.pallas import tpu_sc as plsc

N = 10000
NHID = 256
NCLASS = 40
NLAYERS = 4
E = 160000
DT = 1.0
ALPHA = 1.0
GAMMA = 1.0

NC = 2
NS = 16
EPS = E // NS
CHUNK = 128
NCHUNK = -(-EPS // CHUNK) + (-(-EPS // CHUNK) % 2)
EPAD = NCHUNK * CHUNK
NPASS = 2
CPASS = NCHUNK // NPASS
TRASH = N
ROWS_PER_SUB = 632
NROWS = NS * ROWS_PER_SUB
DUMP_PER_SUB = 624
DUMP_TAIL = N - NS * DUMP_PER_SUB

BM = 1000
GRID = N // BM


def _sc_mesh():
    return plsc.VectorSubcoreMesh(core_axis_name="c", subcore_axis_name="s",
                                  num_cores=NC, num_subcores=NS)


@functools.partial(
    pl.kernel,
    out_type=jax.ShapeDtypeStruct((NC, N, 128), jnp.float32),
    mesh=_sc_mesh(),
    scratch_types=[
        pltpu.VMEM((CPASS, CHUNK), jnp.int32),
        pltpu.VMEM((CHUNK, 128), jnp.float32),
        pltpu.VMEM_SHARED((NROWS, 128), jnp.float32),
        pltpu.SemaphoreType.DMA,
    ],
)
def _deg_sc(dstp_hbm, ones_hbm, zeros_hbm, out_hbm, dst_v, ones_v, acc, sem):
    c = lax.axis_index("c")
    s = lax.axis_index("s")
    pltpu.sync_copy(ones_hbm, ones_v)
    pltpu.sync_copy(zeros_hbm, acc.at[pl.ds(s * ROWS_PER_SUB, ROWS_PER_SUB)])
    plsc.subcore_barrier()
    for p in range(NPASS):
        pltpu.sync_copy(dstp_hbm.at[s, p], dst_v)

        def chunk(j, carry):
            pltpu.sync_copy(ones_v, acc.at[dst_v.at[j]], add=True)
            return carry

        lax.fori_loop(0, CPASS, chunk, 0)
    plsc.subcore_barrier()
    pltpu.sync_copy(acc.at[pl.ds(s * DUMP_PER_SUB, DUMP_PER_SUB)],
                    out_hbm.at[c, pl.ds(s * DUMP_PER_SUB, DUMP_PER_SUB)])

    @pl.when(s == NS - 1)
    def _tail():
        pltpu.sync_copy(acc.at[pl.ds(NS * DUMP_PER_SUB, DUMP_TAIL)],
                        out_hbm.at[c, pl.ds(NS * DUMP_PER_SUB, DUMP_TAIL)])


@functools.partial(
    pl.kernel,
    out_type=jax.ShapeDtypeStruct((NC, N, 128), jnp.float32),
    mesh=_sc_mesh(),
    scratch_types=[
        pltpu.VMEM((CPASS, CHUNK), jnp.int32),
        pltpu.VMEM((CPASS, CHUNK), jnp.int32),
        pltpu.VMEM((CHUNK, 128), jnp.float32),
        pltpu.VMEM((CHUNK, 128), jnp.float32),
        pltpu.VMEM_SHARED((NROWS, 128), jnp.float32),
        pltpu.SemaphoreType.DMA,
        pltpu.SemaphoreType.DMA,
    ],
)
def _agg_sc(xs_hbm, srcp_hbm, dstp_hbm, zeros_hbm, out_hbm,
            src_v, dst_v, buf0, buf1, acc, g0, g1):
    c = lax.axis_index("c")
    s = lax.axis_index("s")
    pltpu.sync_copy(zeros_hbm, acc.at[pl.ds(s * ROWS_PER_SUB, ROWS_PER_SUB)])
    plsc.subcore_barrier()

    def gat(j, buf, sem):
        pltpu.async_copy(xs_hbm.at[src_v.at[j, pl.ds(0, 64)]],
                         buf.at[pl.ds(0, 64)], sem)
        pltpu.async_copy(xs_hbm.at[src_v.at[j, pl.ds(64, 64)]],
                         buf.at[pl.ds(64, 64)], sem)

    def gwait(j, buf, sem):
        pltpu.make_async_copy(xs_hbm.at[src_v.at[j, pl.ds(0, 64)]],
                              buf.at[pl.ds(0, 64)], sem).wait()
        pltpu.make_async_copy(xs_hbm.at[src_v.at[j, pl.ds(64, 64)]],
                              buf.at[pl.ds(64, 64)], sem).wait()

    for p in range(NPASS):
        pltpu.sync_copy(srcp_hbm.at[c, s, p], src_v)
        pltpu.sync_copy(dstp_hbm.at[s, p], dst_v)
        gat(0, buf0, g0)

        def pair(j2, carry):
            j0 = 2 * j2
            gat(j0 + 1, buf1, g1)
            gwait(j0, buf0, g0)
            pltpu.sync_copy(buf0, acc.at[dst_v.at[j0]], add=True)

            @pl.when(j0 + 2 < CPASS)
            def _nxt():
                gat(j0 + 2, buf0, g0)

            gwait(j0 + 1, buf1, g1)
            pltpu.sync_copy(buf1, acc.at[dst_v.at[j0 + 1]], add=True)
            return carry

        lax.fori_loop(0, CPASS // 2, pair, 0)
    plsc.subcore_barrier()
    pltpu.sync_copy(acc.at[pl.ds(s * DUMP_PER_SUB, DUMP_PER_SUB)],
                    out_hbm.at[c, pl.ds(s * DUMP_PER_SUB, DUMP_PER_SUB)])

    @pl.when(s == NS - 1)
    def _tail():
        pltpu.sync_copy(acc.at[pl.ds(NS * DUMP_PER_SUB, DUMP_TAIL)],
                        out_hbm.at[c, pl.ds(NS * DUMP_PER_SUB, DUMP_TAIL)])


def _dinv_body(degp_ref, o_ref):
    deg = jnp.sum(degp_ref[...], axis=(0, 2)) * (1.0 / (NC * 128.0)) + 1.0
    o_ref[...] = lax.rsqrt(deg)[:, None]


def _mmu_body(x_ref, w_ref, xw_ref):
    xw = jnp.dot(x_ref[...], w_ref[...], preferred_element_type=jnp.float32)
    xw_ref[0] = xw[:, :128]
    xw_ref[1] = xw[:, 128:]


def _scale_body(xw_ref, dinv_ref, xs_ref):
    xs_ref[0] = xw_ref[0] * dinv_ref[...]
    xs_ref[1] = xw_ref[1] * dinv_ref[...]


def _mm_body(x_ref, w_ref, dinv_ref, xs_ref):
    xw = jnp.dot(x_ref[...], w_ref[...], preferred_element_type=jnp.float32)
    xs = xw * dinv_ref[...]
    xs_ref[0] = xs[:, :128]
    xs_ref[1] = xs[:, 128:]


def _updmm_body(p_ref, xs_ref, dinv_ref, b_ref, x_ref, y_ref, w_ref,
                xn_ref, yn_ref, xsn_ref):
    a0 = p_ref[0] + xs_ref[0]
    a1 = p_ref[1] + xs_ref[1]
    conv = jnp.concatenate([a0, a1], axis=1) * dinv_ref[...]
    conv = conv + b_ref[...][None, :]
    relu = jnp.maximum(conv, 0.0)
    x = x_ref[...]
    y = y_ref[...]
    yn = y + DT * (relu - ALPHA * y - GAMMA * x)
    xn = x + DT * yn
    yn_ref[...] = yn
    xn_ref[...] = xn
    xw = jnp.dot(xn, w_ref[...], preferred_element_type=jnp.float32)
    xsn = xw * dinv_ref[...]
    xsn_ref[0] = xsn[:, :128]
    xsn_ref[1] = xsn[:, 128:]


def _upd_body(p_ref, xs_ref, dinv_ref, b_ref, x_ref, y_ref, xn_ref, yn_ref):
    a0 = p_ref[0] + xs_ref[0]
    a1 = p_ref[1] + xs_ref[1]
    conv = jnp.concatenate([a0, a1], axis=1) * dinv_ref[...]
    conv = conv + b_ref[...][None, :]
    relu = jnp.maximum(conv, 0.0)
    x = x_ref[...]
    y = y_ref[...]
    yn = y + DT * (relu - ALPHA * y - GAMMA * x)
    yn_ref[...] = yn
    xn_ref[...] = x + DT * yn


def _ro_body(x_ref, w_ref, b_ref, o_ref):
    o_ref[...] = (jnp.dot(x_ref[...], w_ref[...],
                          preferred_element_type=jnp.float32)
                  + b_ref[...][None, :])


def _dinv_tc(degp):
    return pl.pallas_call(
        _dinv_body,
        grid=(GRID,),
        in_specs=[pl.BlockSpec((NC, BM, 128), lambda i: (0, i, 0))],
        out_specs=pl.BlockSpec((BM, 1), lambda i: (i, 0)),
        out_shape=jax.ShapeDtypeStruct((N, 1), jnp.float32),
    )(degp)


def _mmu_tc(x, w):
    return pl.pallas_call(
        _mmu_body,
        grid=(GRID,),
        in_specs=[
            pl.BlockSpec((BM, NHID), lambda i: (i, 0)),
            pl.BlockSpec((NHID, NHID), lambda i: (0, 0)),
        ],
        out_specs=pl.BlockSpec((2, BM, 128), lambda i: (0, i, 0)),
        out_shape=jax.ShapeDtypeStruct((2, N, 128), jnp.float32),
    )(x, w)


def _scale_tc(xw, dinv):
    return pl.pallas_call(
        _scale_body,
        grid=(GRID,),
        in_specs=[
            pl.BlockSpec((2, BM, 128), lambda i: (0, i, 0)),
            pl.BlockSpec((BM, 1), lambda i: (i, 0)),
        ],
        out_specs=pl.BlockSpec((2, BM, 128), lambda i: (0, i, 0)),
        out_shape=jax.ShapeDtypeStruct((2, N, 128), jnp.float32),
    )(xw, dinv)


def _mm_tc(x, w, dinv):
    return pl.pallas_call(
        _mm_body,
        grid=(GRID,),
        in_specs=[
            pl.BlockSpec((BM, NHID), lambda i: (i, 0)),
            pl.BlockSpec((NHID, NHID), lambda i: (0, 0)),
            pl.BlockSpec((BM, 1), lambda i: (i, 0)),
        ],
        out_specs=pl.BlockSpec((2, BM, 128), lambda i: (0, i, 0)),
        out_shape=jax.ShapeDtypeStruct((2, N, 128), jnp.float32),
    )(x, w, dinv)


def _updmm_tc(p, xs, dinv, b, x, y, w):
    return pl.pallas_call(
        _updmm_body,
        grid=(GRID,),
        in_specs=[
            pl.BlockSpec((NC, BM, 128), lambda i: (0, i, 0)),
            pl.BlockSpec((2, BM, 128), lambda i: (0, i, 0)),
            pl.BlockSpec((BM, 1), lambda i: (i, 0)),
            pl.BlockSpec((NHID,), lambda i: (0,)),
            pl.BlockSpec((BM, NHID), lambda i: (i, 0)),
            pl.BlockSpec((BM, NHID), lambda i: (i, 0)),
            pl.BlockSpec((NHID, NHID), lambda i: (0, 0)),
        ],
        out_specs=[
            pl.BlockSpec((BM, NHID), lambda i: (i, 0)),
            pl.BlockSpec((BM, NHID), lambda i: (i, 0)),
            pl.BlockSpec((2, BM, 128), lambda i: (0, i, 0)),
        ],
        out_shape=[jax.ShapeDtypeStruct((N, NHID), jnp.float32),
                   jax.ShapeDtypeStruct((N, NHID), jnp.float32),
                   jax.ShapeDtypeStruct((2, N, 128), jnp.float32)],
    )(p, xs, dinv, b, x, y, w)


def _upd_tc(p, xs, dinv, b, x, y):
    return pl.pallas_call(
        _upd_body,
        grid=(GRID,),
        in_specs=[
            pl.BlockSpec((NC, BM, 128), lambda i: (0, i, 0)),
            pl.BlockSpec((2, BM, 128), lambda i: (0, i, 0)),
            pl.BlockSpec((BM, 1), lambda i: (i, 0)),
            pl.BlockSpec((NHID,), lambda i: (0,)),
            pl.BlockSpec((BM, NHID), lambda i: (i, 0)),
            pl.BlockSpec((BM, NHID), lambda i: (i, 0)),
        ],
        out_specs=[
            pl.BlockSpec((BM, NHID), lambda i: (i, 0)),
            pl.BlockSpec((BM, NHID), lambda i: (i, 0)),
        ],
        out_shape=[jax.ShapeDtypeStruct((N, NHID), jnp.float32),
                   jax.ShapeDtypeStruct((N, NHID), jnp.float32)],
    )(p, xs, dinv, b, x, y)


def _ro_tc(x, w, b):
    return pl.pallas_call(
        _ro_body,
        grid=(GRID,),
        in_specs=[
            pl.BlockSpec((BM, NHID), lambda i: (i, 0)),
            pl.BlockSpec((NHID, 128), lambda i: (0, 0)),
            pl.BlockSpec((128,), lambda i: (0,)),
        ],
        out_specs=pl.BlockSpec((BM, 128), lambda i: (i, 0)),
        out_shape=jax.ShapeDtypeStruct((N, 128), jnp.float32),
    )(x, w, b)


def kernel(x, edge_index, W_conv, b_conv, W_read, b_read):
    src = edge_index[0]
    dst = edge_index[1]
    src16 = jnp.pad(src.reshape(NS, EPS), ((0, 0), (0, EPAD - EPS))
                    ).reshape(NS, NPASS, CPASS, CHUNK)
    srcp = jnp.stack([src16, src16 + N])
    dstp = jnp.pad(dst.reshape(NS, EPS), ((0, 0), (0, EPAD - EPS)),
                   constant_values=TRASH).reshape(NS, NPASS, CPASS, CHUNK)
    ones128 = jnp.ones((CHUNK, 128), jnp.float32)
    zrows = jnp.zeros((ROWS_PER_SUB, 128), jnp.float32)
    Wr = jnp.zeros((NHID, 128), jnp.float32).at[:, :NCLASS].set(W_read)
    br = jnp.zeros((128,), jnp.float32).at[:NCLASS].set(b_read)

    degp = _deg_sc(dstp, ones128, zrows)
    xw1 = _mmu_tc(x, W_conv)
    dinv = _dinv_tc(degp)

    X = x
    Y = x
    Xs = [x]
    Ys = [x]
    xs = _scale_tc(xw1, dinv)
    for layer in range(NLAYERS):
        p = _agg_sc(xs.reshape(2 * N, 128), srcp, dstp, zrows)
        if layer < NLAYERS - 1:
            X, Y, xs = _updmm_tc(p, xs, dinv, b_conv, X, Y, W_conv)
        else:
            X, Y = _upd_tc(p, xs, dinv, b_conv, X, Y)
        Xs.append(X)
        Ys.append(Y)

    logits = _ro_tc(X, Wr, br)[:, :NCLASS]
    return (logits, jnp.stack(Xs, axis=1), jnp.stack(Ys, axis=1))


# speedup vs baseline: 1.0077x; 1.0077x over previous
import functools

import jax
import jax.numpy as jnp
from jax import lax
from jax.experimental import pallas as pl
from jax.experimental.pallas import tpu as pltpu
from jax.experimental.pallas import tpu_sc as plsc

N = 10000
NHID = 256
NCLASS = 40
NLAYERS = 4
E = 160000
DT = 1.0
ALPHA = 1.0
GAMMA = 1.0

NC = 2
NS = 16
EPS = E // NS
CHUNK = 128
NCHUNK = -(-EPS // CHUNK) + (-(-EPS // CHUNK) % 2)
EPAD = NCHUNK * CHUNK
NPASS = 2
CPASS = NCHUNK // NPASS
TRASH = N
ROWS_PER_SUB = 632
NROWS = NS * ROWS_PER_SUB
DUMP_PER_SUB = 624
DUMP_TAIL = N - NS * DUMP_PER_SUB

BM = 1000
GRID = N // BM


def _sc_mesh():
    return plsc.VectorSubcoreMesh(core_axis_name="c", subcore_axis_name="s",
                                  num_cores=NC, num_subcores=NS)


@functools.partial(
    pl.kernel,
    out_type=jax.ShapeDtypeStruct((NC, N, 128), jnp.float32),
    mesh=_sc_mesh(),
    scratch_types=[
        pltpu.VMEM((CPASS, CHUNK), jnp.int32),
        pltpu.VMEM((CHUNK, 128), jnp.float32),
        pltpu.VMEM_SHARED((NROWS, 128), jnp.float32),
        pltpu.SemaphoreType.DMA,
    ],
)
def _deg_sc(dstp_hbm, ones_hbm, zeros_hbm, out_hbm, dst_v, ones_v, acc, sem):
    c = lax.axis_index("c")
    s = lax.axis_index("s")
    pltpu.sync_copy(ones_hbm, ones_v)
    pltpu.sync_copy(zeros_hbm, acc.at[pl.ds(s * ROWS_PER_SUB, ROWS_PER_SUB)])
    plsc.subcore_barrier()
    for p in range(NPASS):
        pltpu.sync_copy(dstp_hbm.at[s, p], dst_v)

        def chunk(j, carry):
            pltpu.sync_copy(ones_v, acc.at[dst_v.at[j]], add=True)
            return carry

        lax.fori_loop(0, CPASS, chunk, 0)
    plsc.subcore_barrier()
    pltpu.sync_copy(acc.at[pl.ds(s * DUMP_PER_SUB, DUMP_PER_SUB)],
                    out_hbm.at[c, pl.ds(s * DUMP_PER_SUB, DUMP_PER_SUB)])

    @pl.when(s == NS - 1)
    def _tail():
        pltpu.sync_copy(acc.at[pl.ds(NS * DUMP_PER_SUB, DUMP_TAIL)],
                        out_hbm.at[c, pl.ds(NS * DUMP_PER_SUB, DUMP_TAIL)])


@functools.partial(
    pl.kernel,
    out_type=jax.ShapeDtypeStruct((NC, N, 128), jnp.float32),
    mesh=_sc_mesh(),
    scratch_types=[
        pltpu.VMEM((CPASS, CHUNK), jnp.int32),
        pltpu.VMEM((CPASS, CHUNK), jnp.int32),
        pltpu.VMEM((CHUNK, 128), jnp.float32),
        pltpu.VMEM((CHUNK, 128), jnp.float32),
        pltpu.VMEM_SHARED((NROWS, 128), jnp.float32),
        pltpu.SemaphoreType.DMA,
        pltpu.SemaphoreType.DMA,
    ],
)
def _agg_sc(xs_hbm, srcp_hbm, dstp_hbm, zeros_hbm, out_hbm,
            src_v, dst_v, buf0, buf1, acc, g0, g1):
    c = lax.axis_index("c")
    s = lax.axis_index("s")
    pltpu.sync_copy(zeros_hbm, acc.at[pl.ds(s * ROWS_PER_SUB, ROWS_PER_SUB)])
    plsc.subcore_barrier()

    def gat(j, buf, sem):
        pltpu.async_copy(xs_hbm.at[src_v.at[j, pl.ds(0, 64)]],
                         buf.at[pl.ds(0, 64)], sem)
        pltpu.async_copy(xs_hbm.at[src_v.at[j, pl.ds(64, 64)]],
                         buf.at[pl.ds(64, 64)], sem)

    def gwait(j, buf, sem):
        pltpu.make_async_copy(xs_hbm.at[src_v.at[j, pl.ds(0, 64)]],
                              buf.at[pl.ds(0, 64)], sem).wait()
        pltpu.make_async_copy(xs_hbm.at[src_v.at[j, pl.ds(64, 64)]],
                              buf.at[pl.ds(64, 64)], sem).wait()

    for p in range(NPASS):
        pltpu.sync_copy(srcp_hbm.at[c, s, p], src_v)
        pltpu.sync_copy(dstp_hbm.at[s, p], dst_v)
        gat(0, buf0, g0)

        def pair(j2, carry):
            j0 = 2 * j2
            gat(j0 + 1, buf1, g1)
            gwait(j0, buf0, g0)
            pltpu.sync_copy(buf0, acc.at[dst_v.at[j0]], add=True)

            @pl.when(j0 + 2 < CPASS)
            def _nxt():
                gat(j0 + 2, buf0, g0)

            gwait(j0 + 1, buf1, g1)
            pltpu.sync_copy(buf1, acc.at[dst_v.at[j0 + 1]], add=True)
            return carry

        lax.fori_loop(0, CPASS // 2, pair, 0)
    plsc.subcore_barrier()
    pltpu.sync_copy(acc.at[pl.ds(s * DUMP_PER_SUB, DUMP_PER_SUB)],
                    out_hbm.at[c, pl.ds(s * DUMP_PER_SUB, DUMP_PER_SUB)])

    @pl.when(s == NS - 1)
    def _tail():
        pltpu.sync_copy(acc.at[pl.ds(NS * DUMP_PER_SUB, DUMP_TAIL)],
                        out_hbm.at[c, pl.ds(NS * DUMP_PER_SUB, DUMP_TAIL)])


def _dinv_body(degp_ref, o_ref):
    deg = jnp.sum(degp_ref[...], axis=(0, 2)) * (1.0 / (NC * 128.0)) + 1.0
    o_ref[...] = lax.rsqrt(deg)[:, None]


def _mm_body(x_ref, w_ref, dinv_ref, xs_ref):
    xw = jnp.dot(x_ref[...], w_ref[...], preferred_element_type=jnp.float32)
    xs = xw * dinv_ref[...]
    xs_ref[0] = xs[:, :128]
    xs_ref[1] = xs[:, 128:]


def _updmm_body(p_ref, xs_ref, dinv_ref, b_ref, x_ref, y_ref, w_ref,
                xn_ref, yn_ref, xsn_ref):
    a0 = p_ref[0] + xs_ref[0]
    a1 = p_ref[1] + xs_ref[1]
    conv = jnp.concatenate([a0, a1], axis=1) * dinv_ref[...]
    conv = conv + b_ref[...][None, :]
    relu = jnp.maximum(conv, 0.0)
    x = x_ref[...]
    y = y_ref[...]
    yn = y + DT * (relu - ALPHA * y - GAMMA * x)
    xn = x + DT * yn
    yn_ref[...] = yn
    xn_ref[...] = xn
    xw = jnp.dot(xn, w_ref[...], preferred_element_type=jnp.float32)
    xsn = xw * dinv_ref[...]
    xsn_ref[0] = xsn[:, :128]
    xsn_ref[1] = xsn[:, 128:]


def _updro_body(p_ref, xs_ref, dinv_ref, b_ref, x_ref, y_ref, wr_ref, br_ref,
                xn_ref, yn_ref, lg_ref):
    a0 = p_ref[0] + xs_ref[0]
    a1 = p_ref[1] + xs_ref[1]
    conv = jnp.concatenate([a0, a1], axis=1) * dinv_ref[...]
    conv = conv + b_ref[...][None, :]
    relu = jnp.maximum(conv, 0.0)
    x = x_ref[...]
    y = y_ref[...]
    yn = y + DT * (relu - ALPHA * y - GAMMA * x)
    xn = x + DT * yn
    yn_ref[...] = yn
    xn_ref[...] = xn
    lg_ref[...] = (jnp.dot(xn, wr_ref[...], preferred_element_type=jnp.float32)
                   + br_ref[...][None, :])


def _upd_body(p_ref, xs_ref, dinv_ref, b_ref, x_ref, y_ref, xn_ref, yn_ref):
    a0 = p_ref[0] + xs_ref[0]
    a1 = p_ref[1] + xs_ref[1]
    conv = jnp.concatenate([a0, a1], axis=1) * dinv_ref[...]
    conv = conv + b_ref[...][None, :]
    relu = jnp.maximum(conv, 0.0)
    x = x_ref[...]
    y = y_ref[...]
    yn = y + DT * (relu - ALPHA * y - GAMMA * x)
    yn_ref[...] = yn
    xn_ref[...] = x + DT * yn


def _ro_body(x_ref, w_ref, b_ref, o_ref):
    o_ref[...] = (jnp.dot(x_ref[...], w_ref[...],
                          preferred_element_type=jnp.float32)
                  + b_ref[...][None, :])


def _dinv_tc(degp):
    return pl.pallas_call(
        _dinv_body,
        grid=(GRID,),
        in_specs=[pl.BlockSpec((NC, BM, 128), lambda i: (0, i, 0))],
        out_specs=pl.BlockSpec((BM, 1), lambda i: (i, 0)),
        out_shape=jax.ShapeDtypeStruct((N, 1), jnp.float32),
    )(degp)


def _mm_tc(x, w, dinv):
    return pl.pallas_call(
        _mm_body,
        grid=(GRID,),
        in_specs=[
            pl.BlockSpec((BM, NHID), lambda i: (i, 0)),
            pl.BlockSpec((NHID, NHID), lambda i: (0, 0)),
            pl.BlockSpec((BM, 1), lambda i: (i, 0)),
        ],
        out_specs=pl.BlockSpec((2, BM, 128), lambda i: (0, i, 0)),
        out_shape=jax.ShapeDtypeStruct((2, N, 128), jnp.float32),
    )(x, w, dinv)


def _updmm_tc(p, xs, dinv, b, x, y, w):
    return pl.pallas_call(
        _updmm_body,
        grid=(GRID,),
        in_specs=[
            pl.BlockSpec((NC, BM, 128), lambda i: (0, i, 0)),
            pl.BlockSpec((2, BM, 128), lambda i: (0, i, 0)),
            pl.BlockSpec((BM, 1), lambda i: (i, 0)),
            pl.BlockSpec((NHID,), lambda i: (0,)),
            pl.BlockSpec((BM, NHID), lambda i: (i, 0)),
            pl.BlockSpec((BM, NHID), lambda i: (i, 0)),
            pl.BlockSpec((NHID, NHID), lambda i: (0, 0)),
        ],
        out_specs=[
            pl.BlockSpec((BM, NHID), lambda i: (i, 0)),
            pl.BlockSpec((BM, NHID), lambda i: (i, 0)),
            pl.BlockSpec((2, BM, 128), lambda i: (0, i, 0)),
        ],
        out_shape=[jax.ShapeDtypeStruct((N, NHID), jnp.float32),
                   jax.ShapeDtypeStruct((N, NHID), jnp.float32),
                   jax.ShapeDtypeStruct((2, N, 128), jnp.float32)],
    )(p, xs, dinv, b, x, y, w)


def _updro_tc(p, xs, dinv, b, x, y, wr, br):
    return pl.pallas_call(
        _updro_body,
        grid=(GRID,),
        in_specs=[
            pl.BlockSpec((NC, BM, 128), lambda i: (0, i, 0)),
            pl.BlockSpec((2, BM, 128), lambda i: (0, i, 0)),
            pl.BlockSpec((BM, 1), lambda i: (i, 0)),
            pl.BlockSpec((NHID,), lambda i: (0,)),
            pl.BlockSpec((BM, NHID), lambda i: (i, 0)),
            pl.BlockSpec((BM, NHID), lambda i: (i, 0)),
            pl.BlockSpec((NHID, 128), lambda i: (0, 0)),
            pl.BlockSpec((128,), lambda i: (0,)),
        ],
        out_specs=[
            pl.BlockSpec((BM, NHID), lambda i: (i, 0)),
            pl.BlockSpec((BM, NHID), lambda i: (i, 0)),
            pl.BlockSpec((BM, 128), lambda i: (i, 0)),
        ],
        out_shape=[jax.ShapeDtypeStruct((N, NHID), jnp.float32),
                   jax.ShapeDtypeStruct((N, NHID), jnp.float32),
                   jax.ShapeDtypeStruct((N, 128), jnp.float32)],
    )(p, xs, dinv, b, x, y, wr, br)


def _upd_tc(p, xs, dinv, b, x, y):
    return pl.pallas_call(
        _upd_body,
        grid=(GRID,),
        in_specs=[
            pl.BlockSpec((NC, BM, 128), lambda i: (0, i, 0)),
            pl.BlockSpec((2, BM, 128), lambda i: (0, i, 0)),
            pl.BlockSpec((BM, 1), lambda i: (i, 0)),
            pl.BlockSpec((NHID,), lambda i: (0,)),
            pl.BlockSpec((BM, NHID), lambda i: (i, 0)),
            pl.BlockSpec((BM, NHID), lambda i: (i, 0)),
        ],
        out_specs=[
            pl.BlockSpec((BM, NHID), lambda i: (i, 0)),
            pl.BlockSpec((BM, NHID), lambda i: (i, 0)),
        ],
        out_shape=[jax.ShapeDtypeStruct((N, NHID), jnp.float32),
                   jax.ShapeDtypeStruct((N, NHID), jnp.float32)],
    )(p, xs, dinv, b, x, y)


def _ro_tc(x, w, b):
    return pl.pallas_call(
        _ro_body,
        grid=(GRID,),
        in_specs=[
            pl.BlockSpec((BM, NHID), lambda i: (i, 0)),
            pl.BlockSpec((NHID, 128), lambda i: (0, 0)),
            pl.BlockSpec((128,), lambda i: (0,)),
        ],
        out_specs=pl.BlockSpec((BM, 128), lambda i: (i, 0)),
        out_shape=jax.ShapeDtypeStruct((N, 128), jnp.float32),
    )(x, w, b)


def kernel(x, edge_index, W_conv, b_conv, W_read, b_read):
    src = edge_index[0]
    dst = edge_index[1]
    src16 = jnp.pad(src.reshape(NS, EPS), ((0, 0), (0, EPAD - EPS))
                    ).reshape(NS, NPASS, CPASS, CHUNK)
    srcp = jnp.stack([src16, src16 + N])
    dstp = jnp.pad(dst.reshape(NS, EPS), ((0, 0), (0, EPAD - EPS)),
                   constant_values=TRASH).reshape(NS, NPASS, CPASS, CHUNK)
    ones128 = jnp.ones((CHUNK, 128), jnp.float32)
    zrows = jnp.zeros((ROWS_PER_SUB, 128), jnp.float32)
    Wr = jnp.zeros((NHID, 128), jnp.float32).at[:, :NCLASS].set(W_read)
    br = jnp.zeros((128,), jnp.float32).at[:NCLASS].set(b_read)

    degp = _deg_sc(dstp, ones128, zrows)
    dinv = _dinv_tc(degp)

    X = x
    Y = x
    Xs = [x]
    Ys = [x]
    xs = _mm_tc(X, W_conv, dinv)
    for layer in range(NLAYERS):
        p = _agg_sc(xs.reshape(2 * N, 128), srcp, dstp, zrows)
        if layer < NLAYERS - 1:
            X, Y, xs = _updmm_tc(p, xs, dinv, b_conv, X, Y, W_conv)
        else:
            X, Y, lg = _updro_tc(p, xs, dinv, b_conv, X, Y, Wr, br)
        Xs.append(X)
        Ys.append(Y)

    logits = lg[:, :NCLASS]
    return (logits, jnp.stack(Xs, axis=1), jnp.stack(Ys, axis=1))
